# Initial kernel scaffold; baseline (speedup 1.0000x reference)
import jax
import jax.numpy as jnp
from jax.experimental import pallas as pl


def kernel(x, edge_index, W1, b1, W2, b2):
    raise NotImplementedError("write your pallas kernel here")


# speedup vs baseline: 8.9280x; 8.9280x over previous
import functools

import jax
import jax.numpy as jnp
from jax import lax
from jax.experimental import pallas as pl
from jax.experimental.pallas import tpu as pltpu
from jax.experimental.pallas import tpu_sc as plsc

N_NODES = 10000
N_EDGES = 320000
D_IN = 128
D_HID = 64
D_OUT = 40
D_OUT_PAD = 48

NP = 10240
NW = 32
CHUNK = 128
E_PAD = 323584
NCH = E_PAD // (NW * CHUNK)
ROWS_PER_SUB = NP // 16

_DUMMY_DST = N_NODES


def _mm1_body(x_ref, w_ref, b_ref, o_ref):
    o_ref[...] = (
        jnp.dot(x_ref[...], w_ref[...], preferred_element_type=jnp.float32)
        + b_ref[0][None, :]
    )


def _mm1(x_pad, w1t, b1row):
    return pl.pallas_call(
        _mm1_body,
        grid=(NP // 512,),
        in_specs=[
            pl.BlockSpec((512, D_IN), lambda i: (i, 0)),
            pl.BlockSpec((D_IN, D_HID), lambda i: (0, 0)),
            pl.BlockSpec((8, D_HID), lambda i: (0, 0)),
        ],
        out_specs=pl.BlockSpec((512, D_HID), lambda i: (i, 0)),
        out_shape=jax.ShapeDtypeStruct((NP, D_HID), jnp.float32),
    )(x_pad, w1t, b1row)


def _make_edge_agg(D):
    mesh = plsc.VectorSubcoreMesh(core_axis_name="c", subcore_axis_name="s")

    @functools.partial(
        pl.kernel,
        mesh=mesh,
        compiler_params=pltpu.CompilerParams(use_tc_tiling_on_sc=False),
        out_type=jax.ShapeDtypeStruct((2, NP, D), jnp.float32),
        scratch_types=[
            pltpu.VMEM((NCH, CHUNK), jnp.int32),
            pltpu.VMEM((NCH, CHUNK), jnp.int32),
            pltpu.VMEM((CHUNK, D), jnp.float32),
            pltpu.VMEM((CHUNK, D), jnp.float32),
            pltpu.VMEM_SHARED((NP, D), jnp.float32),
            pltpu.SemaphoreType.DMA,
            pltpu.SemaphoreType.DMA,
        ],
    )
    def k(h_hbm, src_hbm, dst_hbm, out_hbm,
          src_v, dst_v, buf_a, buf_b, acc, sem_a, sem_b):
        c = lax.axis_index("c")
        s = lax.axis_index("s")
        w = c * 16 + s

        @pl.loop(0, CHUNK)
        def _(r):
            @pl.loop(0, D, step=16)
            def _(col):
                buf_a[r, pl.ds(col, 16)] = jnp.zeros((16,), jnp.float32)

        base = s * ROWS_PER_SUB

        @pl.loop(0, ROWS_PER_SUB // CHUNK)
        def _(i):
            pltpu.sync_copy(buf_a, acc.at[pl.ds(base + i * CHUNK, CHUNK)])

        pltpu.sync_copy(src_hbm.at[w], src_v)
        pltpu.sync_copy(dst_hbm.at[w], dst_v)
        plsc.subcore_barrier()

        pltpu.async_copy(h_hbm.at[src_v.at[0]], buf_a, sem_a)

        @pl.loop(0, (NCH - 1) // 2)
        def _(p):
            j = p * 2
            pltpu.async_copy(h_hbm.at[src_v.at[j + 1]], buf_b, sem_b)
            pltpu.make_async_copy(h_hbm.at[src_v.at[j]], buf_a, sem_a).wait()
            pltpu.sync_copy(buf_a, acc.at[dst_v.at[j]], add=True)
            pltpu.async_copy(h_hbm.at[src_v.at[j + 2]], buf_a, sem_a)
            pltpu.make_async_copy(
                h_hbm.at[src_v.at[j + 1]], buf_b, sem_b).wait()
            pltpu.sync_copy(buf_b, acc.at[dst_v.at[j + 1]], add=True)

        pltpu.make_async_copy(
            h_hbm.at[src_v.at[NCH - 1]], buf_a, sem_a).wait()
        pltpu.sync_copy(buf_a, acc.at[dst_v.at[NCH - 1]], add=True)
        plsc.subcore_barrier()

        pltpu.sync_copy(
            acc.at[pl.ds(base, ROWS_PER_SUB)],
            out_hbm.at[c, pl.ds(base, ROWS_PER_SUB)],
        )

    return k


def _mid_body(p_ref, w_ref, b_ref, o_ref):
    agg = p_ref[0] + p_ref[1]
    h = jnp.where(agg > 0, agg, jnp.exp(jnp.minimum(agg, 0.0)) - 1.0)
    o_ref[...] = (
        jnp.dot(h, w_ref[...], preferred_element_type=jnp.float32)
        + b_ref[0][None, :]
    )


def _mid(parts, w2t, b2row):
    return pl.pallas_call(
        _mid_body,
        grid=(NP // 512,),
        in_specs=[
            pl.BlockSpec((2, 512, D_HID), lambda i: (0, i, 0)),
            pl.BlockSpec((D_HID, D_OUT_PAD), lambda i: (0, 0)),
            pl.BlockSpec((8, D_OUT_PAD), lambda i: (0, 0)),
        ],
        out_specs=pl.BlockSpec((512, D_OUT_PAD), lambda i: (i, 0)),
        out_shape=jax.ShapeDtypeStruct((NP, D_OUT_PAD), jnp.float32),
    )(parts, w2t, b2row)


def _final_body(p_ref, o_ref):
    logits = (p_ref[0] + p_ref[1])[:, :D_OUT]
    m = jnp.max(logits, axis=1, keepdims=True)
    e = jnp.exp(logits - m)
    lse = jnp.log(jnp.sum(e, axis=1, keepdims=True)) + m
    o_ref[...] = logits - lse


def _final(parts):
    return pl.pallas_call(
        _final_body,
        grid=(NP // 512,),
        in_specs=[pl.BlockSpec((2, 512, D_OUT_PAD), lambda i: (0, i, 0))],
        out_specs=pl.BlockSpec((512, D_OUT), lambda i: (i, 0)),
        out_shape=jax.ShapeDtypeStruct((NP, D_OUT), jnp.float32),
    )(parts)


def kernel(x, edge_index, W1, b1, W2, b2):
    src = edge_index[0].astype(jnp.int32)
    dst = edge_index[1].astype(jnp.int32)
    pad = E_PAD - N_EDGES
    src = jnp.concatenate([src, jnp.zeros((pad,), jnp.int32)])
    dst = jnp.concatenate([dst, jnp.full((pad,), _DUMMY_DST, jnp.int32)])
    src = src.reshape(NW, NCH, CHUNK)
    dst = dst.reshape(NW, NCH, CHUNK)

    x_pad = jnp.pad(x, ((0, NP - N_NODES), (0, 0)))
    w1t = W1.T
    b1row = jnp.tile(b1[None, :], (8, 1))
    w2t = jnp.pad(W2, ((0, D_OUT_PAD - D_OUT), (0, 0))).T
    b2row = jnp.tile(jnp.pad(b2, (0, D_OUT_PAD - D_OUT))[None, :], (8, 1))

    h1 = _mm1(x_pad, w1t, b1row)
    parts1 = _make_edge_agg(D_HID)(h1, src, dst)
    h2 = _mid(parts1, w2t, b2row)
    parts2 = _make_edge_agg(D_OUT_PAD)(h2, src, dst)
    out = _final(parts2)
    return out[:N_NODES]
